# baseline (speedup 1.0000x reference)
import jax
import jax.numpy as jnp
import numpy as np
from jax import lax
from jax.experimental import pallas as pl
from jax.experimental.pallas import tpu as pltpu
from jax.experimental.pallas import tpu_sc as plsc

L = 16
T = 2 ** 19
F = 2
BASE_RES = 16
PER_LEVEL_SCALE = 1.5
N_POINTS = 262144
PRIME_Y = np.uint32(2654435761)
PRIME_Z = np.uint32(805459861)

NC = 2
NS = 16
NW = NC * NS
PTS_PER_W = N_POINTS // NW
B = 256
N_CHUNKS = PTS_PER_W // B

RES = [int(np.floor(BASE_RES * (PER_LEVEL_SCALE ** l))) for l in range(L)]
SCALES = np.array([r - 1 for r in RES], dtype=np.float32)

SRC_ROWS = L * T * F // 8
INT_ROWS = L * T * F // 16
ROWS_PER_LVL16 = T * F // 16
SPAN = 1024
SPANS_PER_W = INT_ROWS // NS // SPAN


def _tec_body(x_hbm, y_hbm, z_hbm, tab_hbm, scales_hbm,
              out_hbm, tabint_hbm,
              x_v, y_v, z_v, scale_v, qA_v, qB_v, subA_v, subB_v,
              rowsA_v, rowsB_v, out_v, pidx0_v, pidx1_v, stage_v,
              semA, semB):
    cid = lax.axis_index("c")
    sid = lax.axis_index("s")
    wid = sid * NC + cid

    iota = lax.iota(jnp.int32, 16)

    pltpu.sync_copy(scales_hbm, scale_v)

    my_int_base = cid * INT_ROWS

    def spanstep(sp, _):
        r0 = sid * (INT_ROWS // NS) + sp * SPAN

        def gen(k, _):
            r = iota + (r0 + k * 16)
            s0 = ((r >> 4) << 5) + (r & 15)
            pidx0_v[pl.ds(k * 16, 16)] = s0
            pidx1_v[pl.ds(k * 16, 16)] = s0 + 16
            return 0

        lax.fori_loop(0, SPAN // 16, gen, 0, unroll=4)

        cp0 = pltpu.async_copy(tab_hbm.at[pidx0_v], stage_v.at[0], semA)
        cp1 = pltpu.async_copy(tab_hbm.at[pidx1_v], stage_v.at[1], semA)
        cp0.wait()
        cp1.wait()
        dst = tabint_hbm.at[pl.ds(my_int_base + r0, SPAN)]
        pltpu.sync_copy(stage_v.at[0], dst.at[:, pl.ds(0, 8)])
        pltpu.sync_copy(stage_v.at[1], dst.at[:, pl.ds(8, 8)])
        return 0

    lax.fori_loop(0, SPANS_PER_W, spanstep, 0)
    plsc.subcore_barrier()

    def sx_of(l):
        if isinstance(l, int):
            return np.float32(SCALES[l])
        return plsc.load_gather(scale_v, [jnp.broadcast_to(l, (16,))])

    def p1_into(l, q_ref, sub_ref):
        sx = sx_of(l)
        lbase = jnp.broadcast_to(l * ROWS_PER_LVL16 + cid * INT_ROWS, (16,))

        def p1(k, _):
            xv = x_v[pl.ds(k * 16, 16)] * sx
            yv = y_v[pl.ds(k * 16, 16)] * sx
            zv = z_v[pl.ds(k * 16, 16)] * sx
            xi = xv.astype(jnp.uint32)
            yi = yv.astype(jnp.uint32)
            zi = zv.astype(jnp.uint32)
            hx0 = xi
            hx1 = xi + np.uint32(1)
            hy0 = yi * PRIME_Y
            hy1 = hy0 + PRIME_Y
            hz0 = zi * PRIME_Z
            hz1 = hz0 + PRIME_Z
            mask = np.uint32(T - 1)
            for c in range(8):
                hx = hx1 if (c & 1) else hx0
                hy = hy1 if (c & 2) else hy0
                hz = hz1 if (c & 4) else hz0
                idx = plsc.bitcast((hx ^ hy ^ hz) & mask, jnp.int32)
                q = lbase + ((idx >> 7) << 4) + ((idx >> 3) & 15)
                sl = pl.ds(c * B + k * 16, 16)
                q_ref[sl] = q
                sub_ref[sl] = idx & 7
            return 0

        lax.fori_loop(0, B // 16, p1, 0, unroll=2)

    def fire(q_ref, rows_ref, sem):
        return [
            pltpu.async_copy(
                tabint_hbm.at[q_ref.at[pl.ds(c * B, B)]],
                rows_ref.at[c], sem)
            for c in range(8)
        ]

    def drain(cps):
        for cp in cps:
            cp.wait()

    def p2_from(l, rows_ref, sub_ref):
        sx = sx_of(l)
        col0 = jnp.broadcast_to(2 * l, (16,))
        col1 = col0 + 1

        def p2(k, _):
            lanes = iota + k * 16
            xe = x_v[pl.ds(k * 16, 16)] * sx
            ye = y_v[pl.ds(k * 16, 16)] * sx
            ze = z_v[pl.ds(k * 16, 16)] * sx
            wx = xe - xe.astype(jnp.uint32).astype(jnp.float32)
            wy = ye - ye.astype(jnp.uint32).astype(jnp.float32)
            wz = ze - ze.astype(jnp.uint32).astype(jnp.float32)
            ox = np.float32(1.0) - wx
            oy = np.float32(1.0) - wy
            oz = np.float32(1.0) - wz
            acc0 = None
            acc1 = None
            for c in range(8):
                tx = wx if (c & 1) else ox
                ty = wy if (c & 2) else oy
                tz = wz if (c & 4) else oz
                wc = tx * ty * tz
                sub = sub_ref[pl.ds(c * B + k * 16, 16)]
                rv0 = plsc.load_gather(rows_ref.at[c], [lanes, sub])
                rv1 = plsc.load_gather(rows_ref.at[c], [lanes, sub + 8])
                t0 = wc * rv0
                t1 = wc * rv1
                acc0 = t0 if acc0 is None else acc0 + t0
                acc1 = t1 if acc1 is None else acc1 + t1
            plsc.store_scatter(out_v, [lanes, col0], acc0)
            plsc.store_scatter(out_v, [lanes, col1], acc1)
            return 0

        lax.fori_loop(0, B // 16, p2, 0, unroll=2)

    def do_chunk(ci, _):
        base = wid * PTS_PER_W + ci * B
        pltpu.sync_copy(x_hbm.at[pl.ds(base, B)], x_v)
        pltpu.sync_copy(y_hbm.at[pl.ds(base, B)], y_v)
        pltpu.sync_copy(z_hbm.at[pl.ds(base, B)], z_v)

        p1_into(0, qA_v, subA_v)

        def pairstep(lp, _):
            l = 2 * lp
            cpsA = fire(qA_v, rowsA_v, semA)
            p1_into(l + 1, qB_v, subB_v)
            drain(cpsA)
            cpsB = fire(qB_v, rowsB_v, semB)
            p2_from(l, rowsA_v, subA_v)
            p1_into(l + 2, qA_v, subA_v)
            drain(cpsB)
            p2_from(l + 1, rowsB_v, subB_v)
            return 0

        lax.fori_loop(0, L // 2 - 1, pairstep, 0)

        cpsA = fire(qA_v, rowsA_v, semA)
        p1_into(L - 1, qB_v, subB_v)
        drain(cpsA)
        cpsB = fire(qB_v, rowsB_v, semB)
        p2_from(L - 2, rowsA_v, subA_v)
        drain(cpsB)
        p2_from(L - 1, rowsB_v, subB_v)

        pltpu.sync_copy(out_v, out_hbm.at[pl.ds(base, B)])
        return 0

    lax.fori_loop(0, N_CHUNKS, do_chunk, 0)


@jax.jit
def kernel(input, parameters):
    x = input[:, 0]
    y = input[:, 1]
    z = input[:, 2]
    tab = parameters.reshape(L, T // 128, 128, F).transpose(0, 1, 3, 2)
    tab = tab.reshape(SRC_ROWS, 8)

    mesh = plsc.VectorSubcoreMesh(
        core_axis_name="c", subcore_axis_name="s",
        num_cores=NC, num_subcores=NS)
    run = pl.kernel(
        _tec_body,
        out_type=(
            jax.ShapeDtypeStruct((N_POINTS, L * F), jnp.float32),
            jax.ShapeDtypeStruct((NC * INT_ROWS, 16), jnp.float32),
        ),
        mesh=mesh,
        compiler_params=pltpu.CompilerParams(
            needs_layout_passes=False, use_tc_tiling_on_sc=False),
        scratch_types=[
            pltpu.VMEM((B,), jnp.float32),
            pltpu.VMEM((B,), jnp.float32),
            pltpu.VMEM((B,), jnp.float32),
            pltpu.VMEM((16,), jnp.float32),
            pltpu.VMEM((8 * B,), jnp.int32),
            pltpu.VMEM((8 * B,), jnp.int32),
            pltpu.VMEM((8 * B,), jnp.int32),
            pltpu.VMEM((8 * B,), jnp.int32),
            pltpu.VMEM((8, B, 16), jnp.float32),
            pltpu.VMEM((8, B, 16), jnp.float32),
            pltpu.VMEM((B, L * F), jnp.float32),
            pltpu.VMEM((SPAN,), jnp.int32),
            pltpu.VMEM((SPAN,), jnp.int32),
            pltpu.VMEM((2, SPAN, 8), jnp.float32),
            pltpu.SemaphoreType.DMA,
            pltpu.SemaphoreType.DMA,
        ],
    )
    enc, _ = run(x, y, z, tab, jnp.asarray(SCALES))
    return enc


# speedup vs baseline: 1.3249x; 1.3249x over previous
import jax
import jax.numpy as jnp
import numpy as np
from jax import lax
from jax.experimental import pallas as pl
from jax.experimental.pallas import tpu as pltpu
from jax.experimental.pallas import tpu_sc as plsc

L = 16
T = 2 ** 19
F = 2
BASE_RES = 16
PER_LEVEL_SCALE = 1.5
N_POINTS = 262144
PRIME_Y = np.uint32(2654435761)
PRIME_Z = np.uint32(805459861)

NC = 2
NS = 16
NW = NC * NS
PTS_PER_W = N_POINTS // NW
B = 256
N_CHUNKS = PTS_PER_W // B

RES = [int(np.floor(BASE_RES * (PER_LEVEL_SCALE ** l))) for l in range(L)]
SCALES = np.array([r - 1 for r in RES], dtype=np.float32)

SRC_ROWS = L * T * F // 8
INT_ROWS = L * T * F // 16
ROWS_PER_LVL16 = T * F // 16
SPAN = 1024
SPANS_PER_W = INT_ROWS // NS // SPAN


def _tec_body(x_hbm, y_hbm, z_hbm, tab_hbm, scales_hbm,
              out_hbm, tabint_hbm,
              x_v, y_v, z_v, scale_v, qA_v, qB_v, subA_v, subB_v,
              rowsA_v, rowsB_v, out_v, pidx0_v, pidx1_v, stage_v,
              semA, semB):
    cid = lax.axis_index("c")
    sid = lax.axis_index("s")
    wid = sid * NC + cid

    iota = lax.iota(jnp.int32, 16)

    pltpu.sync_copy(scales_hbm, scale_v)

    my_int_base = cid * INT_ROWS

    def spanstep(sp, _):
        r0 = sid * (INT_ROWS // NS) + sp * SPAN

        def gen(k, _):
            r = iota + (r0 + k * 16)
            s0 = ((r >> 4) << 5) + (r & 15)
            pidx0_v[pl.ds(k * 16, 16)] = s0
            pidx1_v[pl.ds(k * 16, 16)] = s0 + 16
            return 0

        lax.fori_loop(0, SPAN // 16, gen, 0, unroll=4)

        cp0 = pltpu.async_copy(tab_hbm.at[pidx0_v], stage_v.at[0], semA)
        cp1 = pltpu.async_copy(tab_hbm.at[pidx1_v], stage_v.at[1], semA)
        cp0.wait()
        cp1.wait()
        dst = tabint_hbm.at[pl.ds(my_int_base + r0, SPAN)]
        pltpu.sync_copy(stage_v.at[0], dst.at[:, pl.ds(0, 8)])
        pltpu.sync_copy(stage_v.at[1], dst.at[:, pl.ds(8, 8)])
        return 0

    lax.fori_loop(0, SPANS_PER_W, spanstep, 0)
    plsc.subcore_barrier()

    def sx_of(l):
        if isinstance(l, int):
            return np.float32(SCALES[l])
        return plsc.load_gather(scale_v, [jnp.broadcast_to(l, (16,))])

    def p1_into(l, q_ref, sub_ref):
        sx = sx_of(l)
        lbase = jnp.broadcast_to(l * ROWS_PER_LVL16 + cid * INT_ROWS, (16,))

        def p1(k, _):
            xv = x_v[pl.ds(k * 16, 16)] * sx
            yv = y_v[pl.ds(k * 16, 16)] * sx
            zv = z_v[pl.ds(k * 16, 16)] * sx
            xi = xv.astype(jnp.uint32)
            yi = yv.astype(jnp.uint32)
            zi = zv.astype(jnp.uint32)
            hx0 = xi
            hx1 = xi + np.uint32(1)
            hy0 = yi * PRIME_Y
            hy1 = hy0 + PRIME_Y
            hz0 = zi * PRIME_Z
            hz1 = hz0 + PRIME_Z
            mask = np.uint32(T - 1)
            for c in range(8):
                hx = hx1 if (c & 1) else hx0
                hy = hy1 if (c & 2) else hy0
                hz = hz1 if (c & 4) else hz0
                idx = plsc.bitcast((hx ^ hy ^ hz) & mask, jnp.int32)
                q = lbase + ((idx >> 7) << 4) + ((idx >> 3) & 15)
                sl = pl.ds(c * B + k * 16, 16)
                q_ref[sl] = q
                sub_ref[sl] = idx & 7
            return 0

        lax.fori_loop(0, B // 16, p1, 0, unroll=2)

    def fire(q_ref, rows_ref, sem):
        return [
            pltpu.async_copy(
                tabint_hbm.at[q_ref.at[pl.ds(c * B, B)]],
                rows_ref.at[c], sem)
            for c in range(8)
        ]

    def fire_desc(q_ref, rows_ref, sem):
        return [
            pltpu.make_async_copy(
                tabint_hbm.at[q_ref.at[pl.ds(c * B, B)]],
                rows_ref.at[c], sem)
            for c in range(8)
        ]

    def drain(cps):
        for cp in cps:
            cp.wait()

    def p2_from(l, rows_ref, sub_ref):
        sx = sx_of(l)
        col0 = jnp.broadcast_to(2 * l, (16,))
        col1 = col0 + 1

        def p2(k, _):
            lanes = iota + k * 16
            xe = x_v[pl.ds(k * 16, 16)] * sx
            ye = y_v[pl.ds(k * 16, 16)] * sx
            ze = z_v[pl.ds(k * 16, 16)] * sx
            wx = xe - xe.astype(jnp.uint32).astype(jnp.float32)
            wy = ye - ye.astype(jnp.uint32).astype(jnp.float32)
            wz = ze - ze.astype(jnp.uint32).astype(jnp.float32)
            ox = np.float32(1.0) - wx
            oy = np.float32(1.0) - wy
            oz = np.float32(1.0) - wz
            acc0 = None
            acc1 = None
            for c in range(8):
                tx = wx if (c & 1) else ox
                ty = wy if (c & 2) else oy
                tz = wz if (c & 4) else oz
                wc = tx * ty * tz
                sub = sub_ref[pl.ds(c * B + k * 16, 16)]
                rv0 = plsc.load_gather(rows_ref.at[c], [lanes, sub])
                rv1 = plsc.load_gather(rows_ref.at[c], [lanes, sub + 8])
                t0 = wc * rv0
                t1 = wc * rv1
                acc0 = t0 if acc0 is None else acc0 + t0
                acc1 = t1 if acc1 is None else acc1 + t1
            plsc.store_scatter(out_v, [lanes, col0], acc0)
            plsc.store_scatter(out_v, [lanes, col1], acc1)
            return 0

        lax.fori_loop(0, B // 16, p2, 0, unroll=2)

    def do_chunk(ci, _):
        base = wid * PTS_PER_W + ci * B
        pltpu.sync_copy(x_hbm.at[pl.ds(base, B)], x_v)
        pltpu.sync_copy(y_hbm.at[pl.ds(base, B)], y_v)
        pltpu.sync_copy(z_hbm.at[pl.ds(base, B)], z_v)

        p1_into(0, qA_v, subA_v)
        fire(qA_v, rowsA_v, semA)
        p1_into(1, qB_v, subB_v)
        fire(qB_v, rowsB_v, semB)

        def pairstep(lp, _):
            l = 2 * lp
            drain(fire_desc(qA_v, rowsA_v, semA))
            p2_from(l, rowsA_v, subA_v)
            p1_into(l + 2, qA_v, subA_v)
            fire(qA_v, rowsA_v, semA)
            drain(fire_desc(qB_v, rowsB_v, semB))
            p2_from(l + 1, rowsB_v, subB_v)
            p1_into(l + 3, qB_v, subB_v)
            fire(qB_v, rowsB_v, semB)
            return 0

        lax.fori_loop(0, L // 2 - 2, pairstep, 0)

        drain(fire_desc(qA_v, rowsA_v, semA))
        p2_from(L - 4, rowsA_v, subA_v)
        p1_into(L - 2, qA_v, subA_v)
        fire(qA_v, rowsA_v, semA)
        drain(fire_desc(qB_v, rowsB_v, semB))
        p2_from(L - 3, rowsB_v, subB_v)
        p1_into(L - 1, qB_v, subB_v)
        fire(qB_v, rowsB_v, semB)
        drain(fire_desc(qA_v, rowsA_v, semA))
        p2_from(L - 2, rowsA_v, subA_v)
        drain(fire_desc(qB_v, rowsB_v, semB))
        p2_from(L - 1, rowsB_v, subB_v)

        pltpu.sync_copy(out_v, out_hbm.at[pl.ds(base, B)])
        return 0

    lax.fori_loop(0, N_CHUNKS, do_chunk, 0)


@jax.jit
def kernel(input, parameters):
    x = input[:, 0]
    y = input[:, 1]
    z = input[:, 2]
    tab = parameters.reshape(L, T // 128, 128, F).transpose(0, 1, 3, 2)
    tab = tab.reshape(SRC_ROWS, 8)

    mesh = plsc.VectorSubcoreMesh(
        core_axis_name="c", subcore_axis_name="s",
        num_cores=NC, num_subcores=NS)
    run = pl.kernel(
        _tec_body,
        out_type=(
            jax.ShapeDtypeStruct((N_POINTS, L * F), jnp.float32),
            jax.ShapeDtypeStruct((NC * INT_ROWS, 16), jnp.float32),
        ),
        mesh=mesh,
        compiler_params=pltpu.CompilerParams(
            needs_layout_passes=False, use_tc_tiling_on_sc=False),
        scratch_types=[
            pltpu.VMEM((B,), jnp.float32),
            pltpu.VMEM((B,), jnp.float32),
            pltpu.VMEM((B,), jnp.float32),
            pltpu.VMEM((16,), jnp.float32),
            pltpu.VMEM((8 * B,), jnp.int32),
            pltpu.VMEM((8 * B,), jnp.int32),
            pltpu.VMEM((8 * B,), jnp.int32),
            pltpu.VMEM((8 * B,), jnp.int32),
            pltpu.VMEM((8, B, 16), jnp.float32),
            pltpu.VMEM((8, B, 16), jnp.float32),
            pltpu.VMEM((B, L * F), jnp.float32),
            pltpu.VMEM((SPAN,), jnp.int32),
            pltpu.VMEM((SPAN,), jnp.int32),
            pltpu.VMEM((2, SPAN, 8), jnp.float32),
            pltpu.SemaphoreType.DMA,
            pltpu.SemaphoreType.DMA,
        ],
    )
    enc, _ = run(x, y, z, tab, jnp.asarray(SCALES))
    return enc
